# baseline (speedup 1.0000x reference)
import functools

import jax
import jax.numpy as jnp
from jax.experimental import pallas as pl

N = 8192
F_IN = 128
F_PRIME = 64
NEG_SLOPE = 0.01

ROW_BLOCK = 512
N_CHUNKS = 64


def _proj_kernel(h_ref, wst_ref, bs_ref, wtt_ref, bt_ref, a1_ref, a2_ref, ab_ref,
                 z_ref, P_ref, psm_ref, Q_ref, qsm_ref):
    h = h_ref[...]
    z_src = jnp.dot(h, wst_ref[...], preferred_element_type=jnp.float32) + bs_ref[...]
    z_tar = jnp.dot(h, wtt_ref[...], preferred_element_type=jnp.float32) + bt_ref[...]
    s = jnp.dot(z_src, a1_ref[...], preferred_element_type=jnp.float32) + ab_ref[...]
    t = jnp.dot(z_tar, a2_ref[...], preferred_element_type=jnp.float32)
    z_ref[...] = z_src
    P_ref[...] = jnp.exp(s)
    psm_ref[...] = jnp.exp(NEG_SLOPE * s)
    Q_ref[...] = jnp.exp(t)
    qsm_ref[...] = jnp.exp(NEG_SLOPE * t)


def _agg_kernel(*refs):
    a_refs = refs[:N_CHUNKS]
    P_ref, psm_ref, Q_ref, qsm_ref, z_ref, out_ref = refs[N_CHUNKS:]
    C = N // N_CHUNKS
    P = P_ref[...]
    psm = psm_ref[...]
    den = None
    num = None
    for c in range(N_CHUNKS):
        e = jnp.maximum(P * Q_ref[:, c * C:(c + 1) * C],
                        psm * qsm_ref[:, c * C:(c + 1) * C])
        d = jnp.sum(e * a_refs[c][...].astype(jnp.float32), axis=1, keepdims=True)
        n = jnp.dot(e, z_ref[c * C:(c + 1) * C, :],
                    preferred_element_type=jnp.float32)
        den = d if den is None else den + d
        num = n if num is None else num + n
    out_ref[...] = jax.nn.sigmoid(num / den)


@jax.jit
def kernel(H, A, W_src_w, W_src_b, W_tar_w, W_tar_b, a_w, a_b):
    wst = W_src_w.T
    wtt = W_tar_w.T
    bs = W_src_b.reshape(1, F_PRIME)
    bt = W_tar_b.reshape(1, F_PRIME)
    a1 = a_w[:, :F_PRIME].T
    a2 = a_w[:, F_PRIME:].T
    ab = a_b.reshape(1, 1)

    z_src, P, p_sm, Q, q_sm = pl.pallas_call(
        _proj_kernel,
        out_shape=(
            jax.ShapeDtypeStruct((N, F_PRIME), jnp.float32),
            jax.ShapeDtypeStruct((N, 1), jnp.float32),
            jax.ShapeDtypeStruct((N, 1), jnp.float32),
            jax.ShapeDtypeStruct((N, 1), jnp.float32),
            jax.ShapeDtypeStruct((N, 1), jnp.float32),
        ),
    )(H, wst, bs, wtt, bt, a1, a2, ab)

    Q_row = Q.T
    q_row = q_sm.T

    grid = (N // ROW_BLOCK,)
    out = pl.pallas_call(
        _agg_kernel,
        grid=grid,
        in_specs=[
            pl.BlockSpec((ROW_BLOCK, N // N_CHUNKS),
                         functools.partial(lambda c, i: (i, c), c))
            for c in range(N_CHUNKS)
        ] + [
            pl.BlockSpec((ROW_BLOCK, 1), lambda i: (i, 0)),
            pl.BlockSpec((ROW_BLOCK, 1), lambda i: (i, 0)),
            pl.BlockSpec((1, N), lambda i: (0, 0)),
            pl.BlockSpec((1, N), lambda i: (0, 0)),
            pl.BlockSpec((N, F_PRIME), lambda i: (0, 0)),
        ],
        out_specs=pl.BlockSpec((ROW_BLOCK, F_PRIME), lambda i: (i, 0)),
        out_shape=jax.ShapeDtypeStruct((N, F_PRIME), jnp.float32),
    )(*([A] * N_CHUNKS), P, p_sm, Q_row, q_row, z_src)
    return out


# speedup vs baseline: 1.0414x; 1.0414x over previous
import functools

import jax
import jax.numpy as jnp
from jax.experimental import pallas as pl

N = 8192
F_IN = 128
F_PRIME = 64
NEG_SLOPE = 0.01

ROW_BLOCK = 256
N_CHUNKS = 32


def _proj_kernel(h_ref, wst_ref, bs_ref, wtt_ref, bt_ref, a1_ref, a2_ref, ab_ref,
                 z_ref, P_ref, psm_ref, Q_ref, qsm_ref):
    h = h_ref[...]
    z_src = jnp.dot(h, wst_ref[...], preferred_element_type=jnp.float32) + bs_ref[...]
    z_tar = jnp.dot(h, wtt_ref[...], preferred_element_type=jnp.float32) + bt_ref[...]
    s = jnp.dot(z_src, a1_ref[...], preferred_element_type=jnp.float32) + ab_ref[...]
    t = jnp.dot(z_tar, a2_ref[...], preferred_element_type=jnp.float32)
    z_ref[...] = z_src
    P_ref[...] = jnp.exp(s)
    psm_ref[...] = jnp.exp(NEG_SLOPE * s)
    Q_ref[...] = jnp.exp(t)
    qsm_ref[...] = jnp.exp(NEG_SLOPE * t)


def _agg_kernel(*refs):
    a_refs = refs[:N_CHUNKS]
    P_ref, psm_ref, Q_ref, qsm_ref, z_ref, out_ref = refs[N_CHUNKS:]
    C = N // N_CHUNKS
    P = P_ref[...]
    psm = psm_ref[...]
    den = None
    num = None
    for c in range(N_CHUNKS):
        e = jnp.maximum(P * Q_ref[:, c * C:(c + 1) * C],
                        psm * qsm_ref[:, c * C:(c + 1) * C])
        d = jnp.sum(e * a_refs[c][...].astype(jnp.float32), axis=1, keepdims=True)
        n = jnp.dot(e, z_ref[c * C:(c + 1) * C, :],
                    preferred_element_type=jnp.float32)
        den = d if den is None else den + d
        num = n if num is None else num + n
    out_ref[...] = jax.nn.sigmoid(num / den)


@jax.jit
def kernel(H, A, W_src_w, W_src_b, W_tar_w, W_tar_b, a_w, a_b):
    wst = W_src_w.T
    wtt = W_tar_w.T
    bs = W_src_b.reshape(1, F_PRIME)
    bt = W_tar_b.reshape(1, F_PRIME)
    a1 = a_w[:, :F_PRIME].T
    a2 = a_w[:, F_PRIME:].T
    ab = a_b.reshape(1, 1)

    z_src, P, p_sm, Q, q_sm = pl.pallas_call(
        _proj_kernel,
        out_shape=(
            jax.ShapeDtypeStruct((N, F_PRIME), jnp.float32),
            jax.ShapeDtypeStruct((N, 1), jnp.float32),
            jax.ShapeDtypeStruct((N, 1), jnp.float32),
            jax.ShapeDtypeStruct((N, 1), jnp.float32),
            jax.ShapeDtypeStruct((N, 1), jnp.float32),
        ),
    )(H, wst, bs, wtt, bt, a1, a2, ab)

    Q_row = Q.T
    q_row = q_sm.T

    grid = (N // ROW_BLOCK,)
    out = pl.pallas_call(
        _agg_kernel,
        grid=grid,
        in_specs=[
            pl.BlockSpec((ROW_BLOCK, N // N_CHUNKS),
                         functools.partial(lambda c, i: (i, c), c))
            for c in range(N_CHUNKS)
        ] + [
            pl.BlockSpec((ROW_BLOCK, 1), lambda i: (i, 0)),
            pl.BlockSpec((ROW_BLOCK, 1), lambda i: (i, 0)),
            pl.BlockSpec((1, N), lambda i: (0, 0)),
            pl.BlockSpec((1, N), lambda i: (0, 0)),
            pl.BlockSpec((N, F_PRIME), lambda i: (0, 0)),
        ],
        out_specs=pl.BlockSpec((ROW_BLOCK, F_PRIME), lambda i: (i, 0)),
        out_shape=jax.ShapeDtypeStruct((N, F_PRIME), jnp.float32),
    )(*([A] * N_CHUNKS), P, p_sm, Q_row, q_row, z_src)
    return out


# speedup vs baseline: 1.0944x; 1.0509x over previous
import functools

import jax
import jax.numpy as jnp
from jax.experimental import pallas as pl
from jax.experimental.pallas import tpu as pltpu

N = 8192
F_IN = 128
F_PRIME = 64
NEG_SLOPE = 0.01

ROW_BLOCK = 512
N_CHUNKS = 32


def _proj_kernel(h_ref, wst_ref, bs_ref, wtt_ref, bt_ref, a1_ref, a2_ref, ab_ref,
                 z_ref, P_ref, psm_ref, Q_ref, qsm_ref):
    h = h_ref[...]
    z_src = jnp.dot(h, wst_ref[...], preferred_element_type=jnp.float32) + bs_ref[...]
    z_tar = jnp.dot(h, wtt_ref[...], preferred_element_type=jnp.float32) + bt_ref[...]
    s = jnp.dot(z_src, a1_ref[...], preferred_element_type=jnp.float32) + ab_ref[...]
    t = jnp.dot(z_tar, a2_ref[...], preferred_element_type=jnp.float32)
    z_ref[...] = z_src
    P_ref[...] = jnp.exp(s)
    psm_ref[...] = jnp.exp(NEG_SLOPE * s)
    Q_ref[...] = jnp.exp(t)
    qsm_ref[...] = jnp.exp(NEG_SLOPE * t)


def _agg_kernel(*refs):
    a_refs = refs[:N_CHUNKS]
    P_ref, psm_ref, Q_ref, qsm_ref, z_ref, out_ref = refs[N_CHUNKS:]
    C = N // N_CHUNKS
    P = P_ref[...]
    psm = psm_ref[...]
    den = None
    num = None
    for c in range(N_CHUNKS):
        e = jnp.maximum(P * Q_ref[:, c * C:(c + 1) * C],
                        psm * qsm_ref[:, c * C:(c + 1) * C])
        d = jnp.sum(e * a_refs[c][...].astype(jnp.float32), axis=1, keepdims=True)
        n = jnp.dot(e, z_ref[c * C:(c + 1) * C, :],
                    preferred_element_type=jnp.float32)
        den = d if den is None else den + d
        num = n if num is None else num + n
    out_ref[...] = jax.nn.sigmoid(num / den)


@jax.jit
def kernel(H, A, W_src_w, W_src_b, W_tar_w, W_tar_b, a_w, a_b):
    wst = W_src_w.T
    wtt = W_tar_w.T
    bs = W_src_b.reshape(1, F_PRIME)
    bt = W_tar_b.reshape(1, F_PRIME)
    a1 = a_w[:, :F_PRIME].T
    a2 = a_w[:, F_PRIME:].T
    ab = a_b.reshape(1, 1)

    z_src, P, p_sm, Q, q_sm = pl.pallas_call(
        _proj_kernel,
        out_shape=(
            jax.ShapeDtypeStruct((N, F_PRIME), jnp.float32),
            jax.ShapeDtypeStruct((N, 1), jnp.float32),
            jax.ShapeDtypeStruct((N, 1), jnp.float32),
            jax.ShapeDtypeStruct((N, 1), jnp.float32),
            jax.ShapeDtypeStruct((N, 1), jnp.float32),
        ),
    )(H, wst, bs, wtt, bt, a1, a2, ab)

    Q_row = Q.T
    q_row = q_sm.T

    grid = (N // ROW_BLOCK,)
    out = pl.pallas_call(
        _agg_kernel,
        grid=grid,
        in_specs=[
            pl.BlockSpec((ROW_BLOCK, N // N_CHUNKS),
                         functools.partial(lambda c, i: (i, c), c))
            for c in range(N_CHUNKS)
        ] + [
            pl.BlockSpec((ROW_BLOCK, 1), lambda i: (i, 0)),
            pl.BlockSpec((ROW_BLOCK, 1), lambda i: (i, 0)),
            pl.BlockSpec((1, N), lambda i: (0, 0)),
            pl.BlockSpec((1, N), lambda i: (0, 0)),
            pl.BlockSpec((N, F_PRIME), lambda i: (0, 0)),
        ],
        out_specs=pl.BlockSpec((ROW_BLOCK, F_PRIME), lambda i: (i, 0)),
        out_shape=jax.ShapeDtypeStruct((N, F_PRIME), jnp.float32),
    )(*([A] * N_CHUNKS), P, p_sm, Q_row, q_row, z_src)
    return out
